# baseline (speedup 1.0000x reference)
import functools
import math

import jax
import jax.numpy as jnp
from jax import lax
from jax.experimental import pallas as pl
from jax.experimental.pallas import tpu as pltpu
from jax.experimental.pallas import tpu_sc as plsc

_NC = 2
_NS = 16
_NW = _NC * _NS


def _emb_kernel(T, NB, D, scale):
    UB = 1024
    JU = NB // UB
    n_units = T * JU
    u_per_w = n_units // _NW
    EB = D // 8
    ROWS = EB * (UB // 128) * 8
    PAD = 129
    mesh = plsc.VectorSubcoreMesh(core_axis_name="c", subcore_axis_name="s")

    @functools.partial(
        pl.kernel,
        mesh=mesh,
        compiler_params=pltpu.CompilerParams(
            use_tc_tiling_on_sc=False, needs_layout_passes=False),
        out_type=jax.ShapeDtypeStruct((T, EB * (NB // 128) * 8, 128),
                                      jnp.float32),
        scratch_types=[
            pltpu.VMEM((UB,), jnp.int32),
            pltpu.VMEM((UB, D), jnp.float32),
            pltpu.VMEM((ROWS, PAD), jnp.float32),
            pltpu.SemaphoreType.DMA,
        ],
    )
    def emb_k(tok_hbm, table_hbm, out_hbm, idx_v, rows_v, tbuf_v, sem):
        wid = lax.axis_index("s") * _NC + lax.axis_index("c")
        lane = lax.iota(jnp.int32, 16)
        eight = jnp.full((16,), 8, jnp.int32)
        base_row = (lane // eight) * jnp.full((16,), 64, jnp.int32) \
            + (lane % eight)
        hi_off = jnp.full((16,), 128, jnp.int32)

        def unit_body(ul, carry):
            u = wid * u_per_w + ul
            t = u // JU
            j = u % JU
            toff = pl.multiple_of(t * NB + j * UB, 8)
            pltpu.sync_copy(tok_hbm.at[pl.ds(toff, UB)], idx_v)
            pltpu.async_copy(table_hbm.at[idx_v], rows_v, sem).wait()

            @plsc.parallel_loop(0, UB, unroll=8)
            def trans_body(r):
                row0 = base_row + jnp.full((16,), (r >> 7) << 3, jnp.int32)
                col = jnp.full((16,), r & 127, jnp.int32)
                v0 = rows_v[r, 0:16] * scale
                v1 = rows_v[r, 16:32] * scale
                plsc.store_scatter(tbuf_v, [row0, col], v0)
                plsc.store_scatter(tbuf_v, [row0 + hi_off, col], v1)

            for eb in range(EB):
                pltpu.sync_copy(
                    tbuf_v.at[pl.ds(eb * 64, 64), pl.ds(0, 128)],
                    out_hbm.at[t, pl.ds(eb * (NB // 128) * 8 + j * 64, 64), :])
            return carry

        lax.fori_loop(0, u_per_w, unit_body, 0)

    return emb_k


def _transpose_kernel(V, D):
    VB = 128
    n_blocks = -(-V // VB)
    per_tile = -(-n_blocks // _NW)
    last_v0 = V - VB
    mesh = plsc.VectorSubcoreMesh(core_axis_name="c", subcore_axis_name="s")

    @functools.partial(
        pl.kernel,
        mesh=mesh,
        compiler_params=pltpu.CompilerParams(
            use_tc_tiling_on_sc=False, needs_layout_passes=False),
        out_type=jax.ShapeDtypeStruct((V, D), jnp.float32),
        scratch_types=[
            pltpu.VMEM((D, VB), jnp.float32),
            pltpu.VMEM((VB, D + 1), jnp.float32),
            pltpu.SemaphoreType.DMA,
        ],
    )
    def trans_k(tab_t_hbm, out_hbm, src_v, tbuf_v, sem):
        wid = lax.axis_index("s") * _NC + lax.axis_index("c")
        lane = lax.iota(jnp.int32, 16)

        def blk_body(i, carry):
            blk = i * _NW + wid

            @pl.when(blk < n_blocks)
            def _():
                v0 = pl.multiple_of(jnp.minimum(blk * VB, last_v0), 8)
                pltpu.async_copy(
                    tab_t_hbm.at[:, pl.ds(v0, VB)], src_v, sem).wait()

                @plsc.parallel_loop(0, D * (VB // 16), unroll=8)
                def t_body(c):
                    e = c >> 3
                    l0 = (c & 7) << 4
                    v = src_v[e, pl.ds(l0, 16)]
                    row = jnp.full((16,), l0, jnp.int32) + lane
                    plsc.store_scatter(
                        tbuf_v, [row, jnp.full((16,), e, jnp.int32)], v)

                pltpu.sync_copy(tbuf_v.at[:, pl.ds(0, D)],
                                out_hbm.at[pl.ds(v0, VB), :])
            return carry

        lax.fori_loop(0, per_tile, blk_body, 0)

    return trans_k


def kernel(tokens, table):
    B, T = tokens.shape
    V, D = table.shape
    scale = math.sqrt(D)

    tok_t = tokens.T.reshape(B * T)
    trans_k = _transpose_kernel(V, D)
    tab_lin = trans_k(table.T)
    emb_k = _emb_kernel(T, B, D, scale)
    out5 = emb_k(tok_t, tab_lin)

    out = (out5.reshape(T, D // 8, B // 128, 8, 128)
               .transpose(2, 4, 0, 1, 3)
               .reshape(B, T, D))
    return out


# speedup vs baseline: 4.5412x; 4.5412x over previous
import functools
import math

import jax
import jax.numpy as jnp
from jax import lax
from jax.experimental import pallas as pl
from jax.experimental.pallas import tpu as pltpu
from jax.experimental.pallas import tpu_sc as plsc

_NC = 2
_NS = 16
_NW = _NC * _NS


def _emb_kernel(T, NB, D, scale):
    UB = 1024
    JU = NB // UB
    n_units = T * JU
    u_per_w = n_units // _NW
    EB = D // 8
    ROWS = EB * (UB // 128) * 8
    PAD = 129
    mesh = plsc.VectorSubcoreMesh(core_axis_name="c", subcore_axis_name="s")

    @functools.partial(
        pl.kernel,
        mesh=mesh,
        compiler_params=pltpu.CompilerParams(
            use_tc_tiling_on_sc=False, needs_layout_passes=False),
        out_type=jax.ShapeDtypeStruct((T, EB * (NB // 128) * 8, 128),
                                      jnp.float32),
        scratch_types=[
            pltpu.VMEM((UB,), jnp.int32),
            pltpu.VMEM((UB, D), jnp.float32),
            pltpu.VMEM((ROWS, PAD), jnp.float32),
            pltpu.SemaphoreType.DMA,
        ],
    )
    def emb_k(tok_hbm, table_hbm, out_hbm, idx_v, rows_v, tbuf_v, sem):
        wid = lax.axis_index("s") * _NC + lax.axis_index("c")
        lane = lax.iota(jnp.int32, 16)
        eight = jnp.full((16,), 8, jnp.int32)
        base_row = (lane // eight) * jnp.full((16,), 64, jnp.int32) \
            + (lane % eight)
        hi_off = jnp.full((16,), 128, jnp.int32)

        def unit_body(ul, carry):
            u = wid * u_per_w + ul
            t = u // JU
            j = u % JU
            toff = pl.multiple_of(t * NB + j * UB, 8)
            pltpu.sync_copy(tok_hbm.at[pl.ds(toff, UB)], idx_v)
            pltpu.async_copy(table_hbm.at[idx_v], rows_v, sem).wait()

            @plsc.parallel_loop(0, UB, unroll=8)
            def trans_body(r):
                row0 = base_row + jnp.full((16,), (r >> 7) << 3, jnp.int32)
                col = jnp.full((16,), r & 127, jnp.int32)
                v0 = rows_v[r, 0:16] * scale
                v1 = rows_v[r, 16:32] * scale
                plsc.store_scatter(tbuf_v, [row0, col], v0)
                plsc.store_scatter(tbuf_v, [row0 + hi_off, col], v1)

            for eb in range(EB):
                pltpu.sync_copy(
                    tbuf_v.at[pl.ds(eb * 64, 64), pl.ds(0, 128)],
                    out_hbm.at[t, pl.ds(eb * (NB // 128) * 8 + j * 64, 64), :])
            return carry

        lax.fori_loop(0, u_per_w, unit_body, 0)

    return emb_k


def _transpose_kernel(V, D):
    VB = 128
    n_full = V // VB
    TAIL = V - n_full * VB
    per_tile = -(-(n_full + 1) // _NW)
    mesh = plsc.VectorSubcoreMesh(core_axis_name="c", subcore_axis_name="s")

    @functools.partial(
        pl.kernel,
        mesh=mesh,
        compiler_params=pltpu.CompilerParams(
            use_tc_tiling_on_sc=True, needs_layout_passes=False),
        out_type=jax.ShapeDtypeStruct((V * D,), jnp.float32),
        scratch_types=[
            pltpu.VMEM((D, VB), jnp.float32),
            pltpu.VMEM((VB * D,), jnp.float32),
            pltpu.VMEM((D, TAIL), jnp.float32),
            pltpu.VMEM((TAIL * D,), jnp.float32),
            pltpu.SemaphoreType.DMA,
        ],
    )
    def trans_k(tab_t_hbm, out_hbm, src_v, tbuf_v, tsrc_v, ttbuf_v, sem):
        wid = lax.axis_index("s") * _NC + lax.axis_index("c")
        lane = lax.iota(jnp.int32, 16)

        def diag_pass(src, tbuf, n_lblk):
            def sub_body(s, c2):
                e0 = (s & 1) << 4
                l0 = (s >> 1) << 4
                e_idx = lane + jnp.full((16,), e0, jnp.int32)
                for d in range(16):
                    diag = (lane + d) & 15
                    l_idx = diag + jnp.full((16,), l0, jnp.int32)
                    val = plsc.load_gather(src, [e_idx, l_idx])
                    dst = (l_idx * D) + e_idx
                    plsc.store_scatter(tbuf, [dst], val)
                return c2
            lax.fori_loop(0, (D // 16) * (n_lblk // 16), sub_body, 0)

        def blk_body(i, carry):
            blk = i * _NW + wid

            @pl.when(blk < n_full)
            def _():
                v0 = pl.multiple_of(blk * VB, VB)
                pltpu.async_copy(
                    tab_t_hbm.at[:, pl.ds(v0, VB)], src_v, sem).wait()
                diag_pass(src_v, tbuf_v, VB)
                pltpu.sync_copy(tbuf_v, out_hbm.at[pl.ds(v0 * D, VB * D)])

            @pl.when(blk == n_full)
            def _():
                pltpu.async_copy(
                    tab_t_hbm.at[:, pl.ds(n_full * VB, TAIL)],
                    tsrc_v, sem).wait()
                diag_pass(tsrc_v, ttbuf_v, TAIL)
                pltpu.sync_copy(
                    ttbuf_v, out_hbm.at[pl.ds(n_full * VB * D, TAIL * D)])
            return carry

        lax.fori_loop(0, per_tile, blk_body, 0)

    return trans_k


def kernel(tokens, table):
    B, T = tokens.shape
    V, D = table.shape
    scale = math.sqrt(D)

    tok_t = tokens.T.reshape(B * T)
    trans_k = _transpose_kernel(V, D)
    tab_lin = trans_k(table.T).reshape(V, D)
    emb_k = _emb_kernel(T, B, D, scale)
    out5 = emb_k(tok_t, tab_lin)

    out = (out5.reshape(T, D // 8, B // 128, 8, 128)
               .transpose(2, 4, 0, 1, 3)
               .reshape(B, T, D))
    return out


# speedup vs baseline: 5.8976x; 1.2987x over previous
import functools
import math

import jax
import jax.numpy as jnp
from jax import lax
from jax.experimental import pallas as pl
from jax.experimental.pallas import tpu as pltpu
from jax.experimental.pallas import tpu_sc as plsc

_NC = 2
_NS = 16
_NW = _NC * _NS


def _emb_kernel(T, NB, D, scale):
    UB = 1024
    JU = NB // UB
    n_units = T * JU
    u_per_w = n_units // _NW
    EB = D // 8
    ROWS = EB * (UB // 128) * 8
    PAD = 129
    mesh = plsc.VectorSubcoreMesh(core_axis_name="c", subcore_axis_name="s")

    @functools.partial(
        pl.kernel,
        mesh=mesh,
        compiler_params=pltpu.CompilerParams(
            use_tc_tiling_on_sc=False, needs_layout_passes=False),
        out_type=jax.ShapeDtypeStruct((T, EB * (NB // 128) * 8, 128),
                                      jnp.float32),
        scratch_types=[
            pltpu.VMEM((UB,), jnp.int32),
            pltpu.VMEM((UB, D), jnp.float32),
            pltpu.VMEM((ROWS, PAD), jnp.float32),
            pltpu.SemaphoreType.DMA,
        ],
    )
    def emb_k(tok_hbm, table_hbm, out_hbm, idx_v, rows_v, tbuf_v, sem):
        wid = lax.axis_index("s") * _NC + lax.axis_index("c")
        lane = lax.iota(jnp.int32, 16)
        eight = jnp.full((16,), 8, jnp.int32)
        base_row = (lane // eight) * jnp.full((16,), 64, jnp.int32) \
            + (lane % eight)
        hi_off = jnp.full((16,), 128, jnp.int32)

        def unit_body(ul, carry):
            u = wid * u_per_w + ul
            t = u // JU
            j = u % JU
            toff = pl.multiple_of(t * NB + j * UB, 8)
            pltpu.sync_copy(tok_hbm.at[pl.ds(toff, UB)], idx_v)
            pltpu.async_copy(table_hbm.at[idx_v], rows_v, sem).wait()

            @plsc.parallel_loop(0, UB, unroll=8)
            def trans_body(r):
                row0 = base_row + jnp.full((16,), (r >> 7) << 3, jnp.int32)
                col = jnp.full((16,), r & 127, jnp.int32)
                v0 = rows_v[r, 0:16] * scale
                v1 = rows_v[r, 16:32] * scale
                plsc.store_scatter(tbuf_v, [row0, col], v0)
                plsc.store_scatter(tbuf_v, [row0 + hi_off, col], v1)

            for eb in range(EB):
                pltpu.sync_copy(
                    tbuf_v.at[pl.ds(eb * 64, 64), pl.ds(0, 128)],
                    out_hbm.at[t, pl.ds(eb * (NB // 128) * 8 + j * 64, 64), :])
            return carry

        lax.fori_loop(0, u_per_w, unit_body, 0)

    return emb_k


def _transpose_kernel(V, D):
    VB = 1024
    n_full = V // VB
    TAIL = V - n_full * VB
    per_tile = -(-(n_full + 1) // _NW)
    mesh = plsc.VectorSubcoreMesh(core_axis_name="c", subcore_axis_name="s")

    @functools.partial(
        pl.kernel,
        mesh=mesh,
        compiler_params=pltpu.CompilerParams(
            use_tc_tiling_on_sc=True, needs_layout_passes=False),
        out_type=jax.ShapeDtypeStruct((V * D,), jnp.float32),
        scratch_types=[
            pltpu.VMEM((D, VB), jnp.float32),
            pltpu.VMEM((VB * D,), jnp.float32),
            pltpu.VMEM((D, TAIL), jnp.float32),
            pltpu.VMEM((TAIL * D,), jnp.float32),
            pltpu.SemaphoreType.DMA,
        ],
    )
    def trans_k(tab_t_hbm, out_hbm, src_v, tbuf_v, tsrc_v, ttbuf_v, sem):
        wid = lax.axis_index("s") * _NC + lax.axis_index("c")
        lane = lax.iota(jnp.int32, 16)

        def diag_pass(src, tbuf, n_lblk):
            def sub_body(s, c2):
                e0 = (s & 1) << 4
                l0 = (s >> 1) << 4
                e_idx = lane + jnp.full((16,), e0, jnp.int32)
                for d in range(16):
                    diag = (lane + d) & 15
                    l_idx = diag + jnp.full((16,), l0, jnp.int32)
                    val = plsc.load_gather(src, [e_idx, l_idx])
                    dst = (l_idx * D) + e_idx
                    plsc.store_scatter(tbuf, [dst], val)
                return c2
            lax.fori_loop(0, (D // 16) * (n_lblk // 16), sub_body, 0)

        def blk_body(i, carry):
            blk = i * _NW + wid

            @pl.when(blk < n_full)
            def _():
                v0 = pl.multiple_of(blk * VB, VB)
                pltpu.async_copy(
                    tab_t_hbm.at[:, pl.ds(v0, VB)], src_v, sem).wait()
                diag_pass(src_v, tbuf_v, VB)
                pltpu.sync_copy(tbuf_v, out_hbm.at[pl.ds(v0 * D, VB * D)])

            @pl.when(blk == n_full)
            def _():
                pltpu.async_copy(
                    tab_t_hbm.at[:, pl.ds(n_full * VB, TAIL)],
                    tsrc_v, sem).wait()
                diag_pass(tsrc_v, ttbuf_v, TAIL)
                pltpu.sync_copy(
                    ttbuf_v, out_hbm.at[pl.ds(n_full * VB * D, TAIL * D)])
            return carry

        lax.fori_loop(0, per_tile, blk_body, 0)

    return trans_k


def kernel(tokens, table):
    B, T = tokens.shape
    V, D = table.shape
    scale = math.sqrt(D)

    tok_t = tokens.T.reshape(B * T)
    trans_k = _transpose_kernel(V, D)
    tab_lin = trans_k(table.T).reshape(V, D)
    emb_k = _emb_kernel(T, B, D, scale)
    out5 = emb_k(tok_t, tab_lin)

    out = (out5.reshape(T, D // 8, B // 128, 8, 128)
               .transpose(2, 4, 0, 1, 3)
               .reshape(B, T, D))
    return out


# speedup vs baseline: 7.2579x; 1.2307x over previous
import functools
import math

import jax
import jax.numpy as jnp
from jax import lax
from jax.experimental import pallas as pl
from jax.experimental.pallas import tpu as pltpu
from jax.experimental.pallas import tpu_sc as plsc

_NC = 2
_NS = 16
_NW = _NC * _NS


def _emb_kernel(T, NB, D, scale):
    UB = 1024
    JU = NB // UB
    n_units = T * JU
    u_per_w = n_units // _NW
    EB = D // 8
    ROWS = EB * (UB // 128) * 8
    PAD = 129
    mesh = plsc.VectorSubcoreMesh(core_axis_name="c", subcore_axis_name="s")

    @functools.partial(
        pl.kernel,
        mesh=mesh,
        compiler_params=pltpu.CompilerParams(
            use_tc_tiling_on_sc=False, needs_layout_passes=False),
        out_type=jax.ShapeDtypeStruct((T, EB * (NB // 128) * 8, 128),
                                      jnp.float32),
        scratch_types=[
            pltpu.VMEM((UB,), jnp.int32),
            pltpu.VMEM((UB, D), jnp.float32),
            pltpu.VMEM((ROWS, PAD), jnp.float32),
            pltpu.SemaphoreType.DMA,
        ],
    )
    def emb_k(tok_hbm, table_hbm, out_hbm, idx_v, rows_v, tbuf_v, sem):
        wid = lax.axis_index("s") * _NC + lax.axis_index("c")
        lane = lax.iota(jnp.int32, 16)
        eight = jnp.full((16,), 8, jnp.int32)
        base_row = (lane // eight) * jnp.full((16,), 64, jnp.int32) \
            + (lane % eight)
        hi_off = jnp.full((16,), 128, jnp.int32)

        def unit_body(ul, carry):
            u = wid * u_per_w + ul
            t = u // JU
            j = u % JU
            toff = pl.multiple_of(t * NB + j * UB, 8)
            pltpu.sync_copy(tok_hbm.at[pl.ds(toff, UB)], idx_v)
            pltpu.async_copy(table_hbm.at[idx_v], rows_v, sem).wait()

            @plsc.parallel_loop(0, UB, unroll=8)
            def trans_body(r):
                row0 = base_row + jnp.full((16,), (r >> 7) << 3, jnp.int32)
                col = jnp.full((16,), r & 127, jnp.int32)
                v0 = rows_v[r, 0:16] * scale
                v1 = rows_v[r, 16:32] * scale
                plsc.store_scatter(tbuf_v, [row0, col], v0)
                plsc.store_scatter(tbuf_v, [row0 + hi_off, col], v1)

            for eb in range(EB):
                pltpu.sync_copy(
                    tbuf_v.at[pl.ds(eb * 64, 64), pl.ds(0, 128)],
                    out_hbm.at[t, pl.ds(eb * (NB // 128) * 8 + j * 64, 64), :])
            return carry

        lax.fori_loop(0, u_per_w, unit_body, 0)

    return emb_k


def _transpose_kernel(V, D):
    VB = 512
    n_even = (V // VB // _NW) * _NW
    u_per_tile = n_even // _NW
    EXTRA0 = n_even * VB
    TAIL0 = EXTRA0 + VB
    TAIL = V - TAIL0
    mesh = plsc.VectorSubcoreMesh(core_axis_name="c", subcore_axis_name="s")

    @functools.partial(
        pl.kernel,
        mesh=mesh,
        compiler_params=pltpu.CompilerParams(
            use_tc_tiling_on_sc=True, needs_layout_passes=False),
        out_type=jax.ShapeDtypeStruct((V * D,), jnp.float32),
        scratch_types=[
            pltpu.VMEM((D, VB), jnp.float32),
            pltpu.VMEM((VB * D,), jnp.float32),
            pltpu.VMEM((D, VB), jnp.float32),
            pltpu.VMEM((VB * D,), jnp.float32),
            pltpu.VMEM((D, TAIL), jnp.float32),
            pltpu.VMEM((TAIL * D,), jnp.float32),
            pltpu.SemaphoreType.DMA,
            pltpu.SemaphoreType.DMA,
            pltpu.SemaphoreType.DMA,
            pltpu.SemaphoreType.DMA,
        ],
    )
    def trans_k(tab_t_hbm, out_hbm, src0, tbuf0, src1, tbuf1,
                tsrc_v, ttbuf_v, gs0, gs1, os0, os1):
        wid = lax.axis_index("s") * _NC + lax.axis_index("c")
        lane = lax.iota(jnp.int32, 16)

        def in_off(u):
            return pl.multiple_of((u * _NW + wid) * VB, VB)

        def issue_in(u, src, gsem):
            return pltpu.async_copy(
                tab_t_hbm.at[:, pl.ds(in_off(u), VB)], src, gsem)

        def drain_in(src, gsem):
            pltpu.make_async_copy(
                tab_t_hbm.at[:, pl.ds(0, VB)], src, gsem).wait()

        def issue_out(u, tbuf, osem):
            pltpu.async_copy(
                tbuf, out_hbm.at[pl.ds(in_off(u) * D, VB * D)], osem)

        def drain_out(tbuf, osem):
            pltpu.make_async_copy(
                out_hbm.at[pl.ds(0, VB * D)], tbuf, osem).wait()

        def diag_pass(src, tbuf, n_lblk):
            def sub_body(s, c2):
                e0 = (s & 1) << 4
                l0 = (s >> 1) << 4
                e_idx = lane + jnp.full((16,), e0, jnp.int32)
                for d in range(16):
                    diag = (lane + d) & 15
                    l_idx = diag + jnp.full((16,), l0, jnp.int32)
                    val = plsc.load_gather(src, [e_idx, l_idx])
                    dst = (l_idx * D) + e_idx
                    plsc.store_scatter(tbuf, [dst], val)
                return c2
            lax.fori_loop(0, (D // 16) * (n_lblk // 16), sub_body, 0)

        issue_in(0, src0, gs0)

        def pair_body(k, carry):
            u0 = 2 * k
            issue_in(u0 + 1, src1, gs1)
            drain_in(src0, gs0)

            @pl.when(k > 0)
            def _():
                drain_out(tbuf0, os0)
            diag_pass(src0, tbuf0, VB)
            issue_out(u0, tbuf0, os0)
            issue_in(u0 + 2, src0, gs0)

            drain_in(src1, gs1)

            @pl.when(k > 0)
            def _():
                drain_out(tbuf1, os1)
            diag_pass(src1, tbuf1, VB)
            issue_out(u0 + 1, tbuf1, os1)
            return carry

        lax.fori_loop(0, (u_per_tile - 1) // 2, pair_body, 0)

        drain_in(src0, gs0)
        drain_out(tbuf0, os0)
        diag_pass(src0, tbuf0, VB)
        issue_out(u_per_tile - 1, tbuf0, os0)
        drain_out(tbuf1, os1)

        @pl.when(wid == 0)
        def _():
            pltpu.async_copy(
                tab_t_hbm.at[:, pl.ds(EXTRA0, VB)], src1, gs1).wait()
            diag_pass(src1, tbuf1, VB)
            pltpu.sync_copy(tbuf1, out_hbm.at[pl.ds(EXTRA0 * D, VB * D)])
            pltpu.async_copy(
                tab_t_hbm.at[:, pl.ds(TAIL0, TAIL)], tsrc_v, gs1).wait()
            diag_pass(tsrc_v, ttbuf_v, TAIL)
            pltpu.sync_copy(ttbuf_v, out_hbm.at[pl.ds(TAIL0 * D, TAIL * D)])

        drain_out(tbuf0, os0)

    return trans_k


def kernel(tokens, table):
    B, T = tokens.shape
    V, D = table.shape
    scale = math.sqrt(D)

    tok_t = tokens.T.reshape(B * T)
    trans_k = _transpose_kernel(V, D)
    tab_lin = trans_k(table.T).reshape(V, D)
    emb_k = _emb_kernel(T, B, D, scale)
    out5 = emb_k(tok_t, tab_lin)

    out = (out5.reshape(T, D // 8, B // 128, 8, 128)
               .transpose(2, 4, 0, 1, 3)
               .reshape(B, T, D))
    return out


# speedup vs baseline: 7.8720x; 1.0846x over previous
import functools
import math

import jax
import jax.numpy as jnp
from jax import lax
from jax.experimental import pallas as pl
from jax.experimental.pallas import tpu as pltpu
from jax.experimental.pallas import tpu_sc as plsc

_NC = 2
_NS = 16
_NW = _NC * _NS


def _emb_kernel(T, NB, D, scale):
    UB = 512
    JU = NB // UB
    n_units = T * JU
    u_per_w = n_units // _NW
    EB = D // 8
    ROWS = EB * (UB // 128) * 8
    PAD = 129
    ERS = (NB // 128) * 8
    mesh = plsc.VectorSubcoreMesh(core_axis_name="c", subcore_axis_name="s")

    @functools.partial(
        pl.kernel,
        mesh=mesh,
        compiler_params=pltpu.CompilerParams(
            use_tc_tiling_on_sc=False, needs_layout_passes=False),
        out_type=jax.ShapeDtypeStruct((T, EB * ERS, 128), jnp.float32),
        scratch_types=[
            pltpu.VMEM((UB,), jnp.int32),
            pltpu.VMEM((UB, D), jnp.float32),
            pltpu.VMEM((ROWS, PAD), jnp.float32),
            pltpu.VMEM((UB,), jnp.int32),
            pltpu.VMEM((UB, D), jnp.float32),
            pltpu.VMEM((ROWS, PAD), jnp.float32),
            pltpu.SemaphoreType.DMA,
            pltpu.SemaphoreType.DMA,
            pltpu.SemaphoreType.DMA,
            pltpu.SemaphoreType.DMA,
        ],
    )
    def emb_k(tok_hbm, table_hbm, out_hbm, idx0, rows0, tb0,
              idx1, rows1, tb1, gs0, gs1, os0, os1):
        wid = lax.axis_index("s") * _NC + lax.axis_index("c")
        lane = lax.iota(jnp.int32, 16)
        eight = jnp.full((16,), 8, jnp.int32)
        base_row = (lane // eight) * jnp.full((16,), 32, jnp.int32) \
            + (lane % eight)
        hi_off = jnp.full((16,), 64, jnp.int32)

        def fetch(ul, idx_b, rows_b, gsem):
            u = wid * u_per_w + ul
            t = u // JU
            j = u % JU
            toff = pl.multiple_of(t * NB + j * UB, 8)
            pltpu.sync_copy(tok_hbm.at[pl.ds(toff, UB)], idx_b)
            pltpu.async_copy(table_hbm.at[idx_b], rows_b, gsem)

        def drain_gather(rows_b, gsem):
            pltpu.make_async_copy(
                table_hbm.at[pl.ds(0, UB)], rows_b, gsem).wait()

        def drain_outs(tb_b, osem):
            for _ in range(EB):
                pltpu.make_async_copy(
                    out_hbm.at[0, pl.ds(0, 32), :],
                    tb_b.at[pl.ds(0, 32), pl.ds(0, 128)], osem).wait()

        def process(ul, rows_b, tb_b, osem):
            u = wid * u_per_w + ul
            t = u // JU
            j = u % JU

            @plsc.parallel_loop(0, UB, unroll=8)
            def trans_body(r):
                row0 = base_row + jnp.full((16,), (r >> 7) << 3, jnp.int32)
                col = jnp.full((16,), r & 127, jnp.int32)
                v0 = rows_b[r, 0:16] * scale
                v1 = rows_b[r, 16:32] * scale
                plsc.store_scatter(tb_b, [row0, col], v0)
                plsc.store_scatter(tb_b, [row0 + hi_off, col], v1)

            for eb in range(EB):
                pltpu.async_copy(
                    tb_b.at[pl.ds(eb * 32, 32), pl.ds(0, 128)],
                    out_hbm.at[t, pl.ds(eb * ERS + j * 32, 32), :], osem)

        fetch(0, idx0, rows0, gs0)

        def pair_body(k, carry):
            ul0 = 2 * k
            fetch(ul0 + 1, idx1, rows1, gs1)
            drain_gather(rows0, gs0)

            @pl.when(k > 0)
            def _():
                drain_outs(tb0, os0)
            process(ul0, rows0, tb0, os0)

            @pl.when(k < u_per_w // 2 - 1)
            def _():
                fetch(ul0 + 2, idx0, rows0, gs0)

            drain_gather(rows1, gs1)

            @pl.when(k > 0)
            def _():
                drain_outs(tb1, os1)
            process(ul0 + 1, rows1, tb1, os1)
            return carry

        lax.fori_loop(0, u_per_w // 2, pair_body, 0)
        drain_outs(tb0, os0)
        drain_outs(tb1, os1)

    return emb_k


def _transpose_kernel(V, D):
    VB = 512
    n_even = (V // VB // _NW) * _NW
    u_per_tile = n_even // _NW
    EXTRA0 = n_even * VB
    TAIL0 = EXTRA0 + VB
    TAIL = V - TAIL0
    mesh = plsc.VectorSubcoreMesh(core_axis_name="c", subcore_axis_name="s")

    @functools.partial(
        pl.kernel,
        mesh=mesh,
        compiler_params=pltpu.CompilerParams(
            use_tc_tiling_on_sc=True, needs_layout_passes=False),
        out_type=jax.ShapeDtypeStruct((V * D,), jnp.float32),
        scratch_types=[
            pltpu.VMEM((D, VB), jnp.float32),
            pltpu.VMEM((VB * D,), jnp.float32),
            pltpu.VMEM((D, VB), jnp.float32),
            pltpu.VMEM((VB * D,), jnp.float32),
            pltpu.VMEM((D, TAIL), jnp.float32),
            pltpu.VMEM((TAIL * D,), jnp.float32),
            pltpu.SemaphoreType.DMA,
            pltpu.SemaphoreType.DMA,
            pltpu.SemaphoreType.DMA,
            pltpu.SemaphoreType.DMA,
        ],
    )
    def trans_k(tab_t_hbm, out_hbm, src0, tbuf0, src1, tbuf1,
                tsrc_v, ttbuf_v, gs0, gs1, os0, os1):
        wid = lax.axis_index("s") * _NC + lax.axis_index("c")
        lane = lax.iota(jnp.int32, 16)

        def in_off(u):
            return pl.multiple_of((u * _NW + wid) * VB, VB)

        def issue_in(u, src, gsem):
            return pltpu.async_copy(
                tab_t_hbm.at[:, pl.ds(in_off(u), VB)], src, gsem)

        def drain_in(src, gsem):
            pltpu.make_async_copy(
                tab_t_hbm.at[:, pl.ds(0, VB)], src, gsem).wait()

        def issue_out(u, tbuf, osem):
            pltpu.async_copy(
                tbuf, out_hbm.at[pl.ds(in_off(u) * D, VB * D)], osem)

        def drain_out(tbuf, osem):
            pltpu.make_async_copy(
                out_hbm.at[pl.ds(0, VB * D)], tbuf, osem).wait()

        def diag_pass(src, tbuf, n_lblk):
            def sub_body(s, c2):
                e0 = (s & 1) << 4
                l0 = (s >> 1) << 4
                e_idx = lane + jnp.full((16,), e0, jnp.int32)
                for d in range(16):
                    diag = (lane + d) & 15
                    l_idx = diag + jnp.full((16,), l0, jnp.int32)
                    val = plsc.load_gather(src, [e_idx, l_idx])
                    dst = (l_idx * D) + e_idx
                    plsc.store_scatter(tbuf, [dst], val)
                return c2
            lax.fori_loop(0, (D // 16) * (n_lblk // 16), sub_body, 0)

        issue_in(0, src0, gs0)

        def pair_body(k, carry):
            u0 = 2 * k
            issue_in(u0 + 1, src1, gs1)
            drain_in(src0, gs0)

            @pl.when(k > 0)
            def _():
                drain_out(tbuf0, os0)
            diag_pass(src0, tbuf0, VB)
            issue_out(u0, tbuf0, os0)
            issue_in(u0 + 2, src0, gs0)

            drain_in(src1, gs1)

            @pl.when(k > 0)
            def _():
                drain_out(tbuf1, os1)
            diag_pass(src1, tbuf1, VB)
            issue_out(u0 + 1, tbuf1, os1)
            return carry

        lax.fori_loop(0, (u_per_tile - 1) // 2, pair_body, 0)

        drain_in(src0, gs0)
        drain_out(tbuf0, os0)
        diag_pass(src0, tbuf0, VB)
        issue_out(u_per_tile - 1, tbuf0, os0)
        drain_out(tbuf1, os1)

        @pl.when(wid == 0)
        def _():
            pltpu.async_copy(
                tab_t_hbm.at[:, pl.ds(EXTRA0, VB)], src1, gs1).wait()
            diag_pass(src1, tbuf1, VB)
            pltpu.sync_copy(tbuf1, out_hbm.at[pl.ds(EXTRA0 * D, VB * D)])
            pltpu.async_copy(
                tab_t_hbm.at[:, pl.ds(TAIL0, TAIL)], tsrc_v, gs1).wait()
            diag_pass(tsrc_v, ttbuf_v, TAIL)
            pltpu.sync_copy(ttbuf_v, out_hbm.at[pl.ds(TAIL0 * D, TAIL * D)])

        drain_out(tbuf0, os0)

    return trans_k


def kernel(tokens, table):
    B, T = tokens.shape
    V, D = table.shape
    scale = math.sqrt(D)

    tok_t = tokens.T.reshape(B * T)
    trans_k = _transpose_kernel(V, D)
    tab_lin = trans_k(table.T).reshape(V, D)
    emb_k = _emb_kernel(T, B, D, scale)
    out5 = emb_k(tok_t, tab_lin)

    out = (out5.reshape(T, D // 8, B // 128, 8, 128)
               .transpose(2, 4, 0, 1, 3)
               .reshape(B, T, D))
    return out


# speedup vs baseline: 12.8854x; 1.6369x over previous
import functools
import math

import jax
import jax.numpy as jnp
from jax import lax
from jax.experimental import pallas as pl
from jax.experimental.pallas import tpu as pltpu
from jax.experimental.pallas import tpu_sc as plsc

_NC = 2
_NS = 16
_NW = _NC * _NS


def _emb_kernel(T, NB, D, scale):
    UB = 512
    JU = NB // UB
    n_units = T * JU
    u_per_w = n_units // _NW
    EB = D // 8
    ROWS = EB * (UB // 128) * 8
    PAD = 129
    ERS = (NB // 128) * 8
    mesh = plsc.VectorSubcoreMesh(core_axis_name="c", subcore_axis_name="s")

    @functools.partial(
        pl.kernel,
        mesh=mesh,
        compiler_params=pltpu.CompilerParams(
            use_tc_tiling_on_sc=False, needs_layout_passes=False),
        out_type=jax.ShapeDtypeStruct((T, EB * ERS, 128), jnp.float32),
        scratch_types=[
            pltpu.VMEM((UB,), jnp.int32),
            pltpu.VMEM((UB, D), jnp.float32),
            pltpu.VMEM((ROWS, PAD), jnp.float32),
            pltpu.VMEM((UB,), jnp.int32),
            pltpu.VMEM((UB, D), jnp.float32),
            pltpu.VMEM((ROWS, PAD), jnp.float32),
            pltpu.SemaphoreType.DMA,
            pltpu.SemaphoreType.DMA,
            pltpu.SemaphoreType.DMA,
            pltpu.SemaphoreType.DMA,
        ],
    )
    def emb_k(tok_hbm, table_hbm, out_hbm, idx0, rows0, tb0,
              idx1, rows1, tb1, gs0, gs1, os0, os1):
        wid = lax.axis_index("s") * _NC + lax.axis_index("c")
        lane = lax.iota(jnp.int32, 16)
        eight = jnp.full((16,), 8, jnp.int32)
        base_row = (lane // eight) * jnp.full((16,), 32, jnp.int32) \
            + (lane % eight)
        hi_off = jnp.full((16,), 64, jnp.int32)

        def fetch(ul, idx_b, rows_b, gsem):
            u = wid * u_per_w + ul
            t = u // JU
            j = u % JU
            toff = pl.multiple_of(t * NB + j * UB, 8)
            pltpu.sync_copy(tok_hbm.at[pl.ds(toff, UB)], idx_b)
            pltpu.async_copy(table_hbm.at[idx_b], rows_b, gsem)

        def drain_gather(rows_b, gsem):
            pltpu.make_async_copy(
                table_hbm.at[pl.ds(0, UB)], rows_b, gsem).wait()

        def drain_outs(tb_b, osem):
            for _ in range(EB):
                pltpu.make_async_copy(
                    out_hbm.at[0, pl.ds(0, 32), :],
                    tb_b.at[pl.ds(0, 32), pl.ds(0, 128)], osem).wait()

        def process(ul, rows_b, tb_b, osem):
            u = wid * u_per_w + ul
            t = u // JU
            j = u % JU

            @plsc.parallel_loop(0, UB, unroll=8)
            def trans_body(r):
                row0 = base_row + jnp.full((16,), (r >> 7) << 3, jnp.int32)
                col = jnp.full((16,), r & 127, jnp.int32)
                v0 = rows_b[r, 0:16] * scale
                v1 = rows_b[r, 16:32] * scale
                plsc.store_scatter(tb_b, [row0, col], v0)
                plsc.store_scatter(tb_b, [row0 + hi_off, col], v1)

            for eb in range(EB):
                pltpu.async_copy(
                    tb_b.at[pl.ds(eb * 32, 32), pl.ds(0, 128)],
                    out_hbm.at[t, pl.ds(eb * ERS + j * 32, 32), :], osem)

        fetch(0, idx0, rows0, gs0)

        def pair_body(k, carry):
            ul0 = 2 * k
            fetch(ul0 + 1, idx1, rows1, gs1)
            drain_gather(rows0, gs0)

            @pl.when(k > 0)
            def _():
                drain_outs(tb0, os0)
            process(ul0, rows0, tb0, os0)

            @pl.when(k < u_per_w // 2 - 1)
            def _():
                fetch(ul0 + 2, idx0, rows0, gs0)

            drain_gather(rows1, gs1)

            @pl.when(k > 0)
            def _():
                drain_outs(tb1, os1)
            process(ul0 + 1, rows1, tb1, os1)
            return carry

        lax.fori_loop(0, u_per_w // 2, pair_body, 0)
        drain_outs(tb0, os0)
        drain_outs(tb1, os1)

    return emb_k


def _transpose_kernel(V, D):
    VB = 512
    n_even = (V // VB // _NW) * _NW
    u_per_tile = n_even // _NW
    EXTRA0 = n_even * VB
    TAIL0 = EXTRA0 + VB
    TAIL = V - TAIL0
    mesh = plsc.VectorSubcoreMesh(core_axis_name="c", subcore_axis_name="s")

    @functools.partial(
        pl.kernel,
        mesh=mesh,
        compiler_params=pltpu.CompilerParams(
            use_tc_tiling_on_sc=True, needs_layout_passes=False),
        out_type=jax.ShapeDtypeStruct((V * D,), jnp.float32),
        scratch_types=[
            pltpu.VMEM((D, VB), jnp.float32),
            pltpu.VMEM((VB * D,), jnp.float32),
            pltpu.VMEM((D, VB), jnp.float32),
            pltpu.VMEM((VB * D,), jnp.float32),
            pltpu.VMEM((D, TAIL), jnp.float32),
            pltpu.VMEM((TAIL * D,), jnp.float32),
            pltpu.SemaphoreType.DMA,
            pltpu.SemaphoreType.DMA,
            pltpu.SemaphoreType.DMA,
            pltpu.SemaphoreType.DMA,
        ],
    )
    def trans_k(tab_t_hbm, out_hbm, src0, tbuf0, src1, tbuf1,
                tsrc_v, ttbuf_v, gs0, gs1, os0, os1):
        wid = lax.axis_index("s") * _NC + lax.axis_index("c")
        lane = lax.iota(jnp.int32, 16)

        def in_off(u):
            return pl.multiple_of((u * _NW + wid) * VB, VB)

        def issue_in(u, src, gsem):
            return pltpu.async_copy(
                tab_t_hbm.at[:, pl.ds(in_off(u), VB)], src, gsem)

        def drain_in(src, gsem):
            pltpu.make_async_copy(
                tab_t_hbm.at[:, pl.ds(0, VB)], src, gsem).wait()

        def issue_out(u, tbuf, osem):
            pltpu.async_copy(
                tbuf, out_hbm.at[pl.ds(in_off(u) * D, VB * D)], osem)

        def drain_out(tbuf, osem):
            pltpu.make_async_copy(
                out_hbm.at[pl.ds(0, VB * D)], tbuf, osem).wait()

        def diag_pass(src, tbuf, n_lblk):
            @plsc.parallel_loop(0, (D // 16) * (n_lblk // 16), unroll=2)
            def sub_body(s):
                e0 = (s & 1) << 4
                l0 = (s >> 1) << 4
                e_idx = lane + jnp.full((16,), e0, jnp.int32)
                for d in range(16):
                    diag = (lane + d) & 15
                    dstc = diag * D + lane
                    l_idx = diag + jnp.full((16,), l0, jnp.int32)
                    val = plsc.load_gather(src, [e_idx, l_idx])
                    dst = dstc + jnp.full((16,), l0 * D + e0, jnp.int32)
                    plsc.store_scatter(tbuf, [dst], val)

        issue_in(0, src0, gs0)

        def pair_body(k, carry):
            u0 = 2 * k
            issue_in(u0 + 1, src1, gs1)
            drain_in(src0, gs0)

            @pl.when(k > 0)
            def _():
                drain_out(tbuf0, os0)
            diag_pass(src0, tbuf0, VB)
            issue_out(u0, tbuf0, os0)
            issue_in(u0 + 2, src0, gs0)

            drain_in(src1, gs1)

            @pl.when(k > 0)
            def _():
                drain_out(tbuf1, os1)
            diag_pass(src1, tbuf1, VB)
            issue_out(u0 + 1, tbuf1, os1)
            return carry

        lax.fori_loop(0, (u_per_tile - 1) // 2, pair_body, 0)

        drain_in(src0, gs0)
        drain_out(tbuf0, os0)
        diag_pass(src0, tbuf0, VB)
        issue_out(u_per_tile - 1, tbuf0, os0)
        drain_out(tbuf1, os1)

        @pl.when(wid == 0)
        def _():
            pltpu.async_copy(
                tab_t_hbm.at[:, pl.ds(EXTRA0, VB)], src1, gs1).wait()
            diag_pass(src1, tbuf1, VB)
            pltpu.sync_copy(tbuf1, out_hbm.at[pl.ds(EXTRA0 * D, VB * D)])
            pltpu.async_copy(
                tab_t_hbm.at[:, pl.ds(TAIL0, TAIL)], tsrc_v, gs1).wait()
            diag_pass(tsrc_v, ttbuf_v, TAIL)
            pltpu.sync_copy(ttbuf_v, out_hbm.at[pl.ds(TAIL0 * D, TAIL * D)])

        drain_out(tbuf0, os0)

    return trans_k


def kernel(tokens, table):
    B, T = tokens.shape
    V, D = table.shape
    scale = math.sqrt(D)

    tok_t = tokens.T.reshape(B * T)
    trans_k = _transpose_kernel(V, D)
    tab_lin = trans_k(table.T).reshape(V, D)
    emb_k = _emb_kernel(T, B, D, scale)
    out5 = emb_k(tok_t, tab_lin)

    out = (out5.reshape(T, D // 8, B // 128, 8, 128)
               .transpose(2, 4, 0, 1, 3)
               .reshape(B, T, D))
    return out
